# Initial kernel scaffold; baseline (speedup 1.0000x reference)
import jax
import jax.numpy as jnp
from jax.experimental import pallas as pl


def kernel(x, edge_index, origin_ids, destination_ids, day_type_ids, time_bucket_ids, day_of_week_ids, mode_ids, W1, b1, W2, b2, W3, b3, day_tab, time_tab, dow_tab, Wt, bt, mode_tab, Wp1, bp1, Wp2, bp2, Wp3, bp3, Wp4, bp4):
    raise NotImplementedError("write your pallas kernel here")


# speedup vs baseline: 20.5784x; 20.5784x over previous
import functools

import jax
import jax.numpy as jnp
from jax import lax
from jax.experimental import pallas as pl
from jax.experimental.pallas import tpu as pltpu
from jax.experimental.pallas import tpu_sc as plsc

NN = 10000
NP = 10240
DD = 128
HH = 64
BB = 16384
NC, NS, CH = 2, 16, 128
NW = NC * NS
KE = 79
KB = BB // (NW * CH)
EPAD = NW * KE * CH
RB = 512


def _sc_mesh():
    return plsc.VectorSubcoreMesh(core_axis_name="c", subcore_axis_name="s",
                                  num_cores=NC, num_subcores=NS)


@functools.cache
def _deg_kernel():
    @functools.partial(
        pl.kernel,
        out_type=jax.ShapeDtypeStruct((NC, NP), jnp.float32),
        mesh=_sc_mesh(),
        compiler_params=pltpu.CompilerParams(use_tc_tiling_on_sc=False),
        scratch_types=[
            pltpu.VMEM((KE, CH), jnp.int32),
            pltpu.VMEM((CH,), jnp.float32),
            pltpu.VMEM_SHARED((NP,), jnp.float32),
        ],
    )
    def body_fn(dst_hbm, zeros_hbm, out_hbm, dst_v, ones_v, acc_sh):
        c = lax.axis_index("c")
        s = lax.axis_index("s")
        w = c * NS + s
        pltpu.sync_copy(dst_hbm.at[w], dst_v)
        for j in range(CH // 16):
            ones_v[pl.ds(j * 16, 16)] = jnp.ones((16,), jnp.float32)

        @pl.when(s == 0)
        def _():
            pltpu.sync_copy(zeros_hbm, acc_sh)

        plsc.subcore_barrier()

        def body(k, carry):
            pltpu.sync_copy(ones_v, acc_sh.at[dst_v.at[k]], add=True)
            return carry

        lax.fori_loop(0, KE, body, 0)
        plsc.subcore_barrier()

        @pl.when(s == 0)
        def _():
            pltpu.sync_copy(acc_sh, out_hbm.at[c])

    return body_fn


@functools.cache
def _agg_kernel():
    @functools.partial(
        pl.kernel,
        out_type=jax.ShapeDtypeStruct((NC, NP, HH), jnp.float32),
        mesh=_sc_mesh(),
        compiler_params=pltpu.CompilerParams(use_tc_tiling_on_sc=False),
        scratch_types=[
            pltpu.VMEM((KE, CH), jnp.int32),
            pltpu.VMEM((KE, CH), jnp.int32),
            pltpu.VMEM((CH, HH), jnp.float32),
            pltpu.VMEM_SHARED((NP, HH), jnp.float32),
            pltpu.SemaphoreType.DMA,
        ],
    )
    def body_fn(yp_hbm, src_hbm, dst_hbm, zeros_hbm, out_hbm,
                src_v, dst_v, rows_v, acc_sh, sem):
        c = lax.axis_index("c")
        s = lax.axis_index("s")
        w = c * NS + s
        pltpu.sync_copy(src_hbm.at[w], src_v)
        pltpu.sync_copy(dst_hbm.at[w], dst_v)

        @pl.when(s == 0)
        def _():
            pltpu.sync_copy(zeros_hbm, acc_sh)

        plsc.subcore_barrier()

        def body(k, carry):
            pltpu.async_copy(yp_hbm.at[src_v.at[k]], rows_v, sem).wait()
            pltpu.sync_copy(rows_v, acc_sh.at[dst_v.at[k]], add=True)
            return carry

        lax.fori_loop(0, KE, body, 0)
        plsc.subcore_barrier()

        @pl.when(s == 0)
        def _():
            pltpu.sync_copy(acc_sh, out_hbm.at[c])

    return body_fn


@functools.cache
def _pair_gather_kernel():
    @functools.partial(
        pl.kernel,
        out_type=[
            jax.ShapeDtypeStruct((BB, 2 * HH), jnp.float32),
            jax.ShapeDtypeStruct((BB, 2 * HH), jnp.float32),
        ],
        mesh=_sc_mesh(),
        compiler_params=pltpu.CompilerParams(use_tc_tiling_on_sc=False),
        scratch_types=[
            pltpu.VMEM((KB, CH), jnp.int32),
            pltpu.VMEM((KB, CH), jnp.int32),
            pltpu.VMEM((CH, 2 * HH), jnp.float32),
            pltpu.SemaphoreType.DMA,
        ],
    )
    def body_fn(go_hbm, gd_hbm, oid_hbm, did_hbm, oo_hbm, od_hbm,
                oi_v, di_v, rows_v, sem):
        c = lax.axis_index("c")
        s = lax.axis_index("s")
        w = c * NS + s
        pltpu.sync_copy(oid_hbm.at[w], oi_v)
        pltpu.sync_copy(did_hbm.at[w], di_v)

        def body(k, carry):
            base = w * (KB * CH) + k * CH
            pltpu.async_copy(go_hbm.at[oi_v.at[k]], rows_v, sem).wait()
            pltpu.sync_copy(rows_v, oo_hbm.at[pl.ds(base, CH)])
            pltpu.async_copy(gd_hbm.at[di_v.at[k]], rows_v, sem).wait()
            pltpu.sync_copy(rows_v, od_hbm.at[pl.ds(base, CH)])
            return carry

        lax.fori_loop(0, KB, body, 0)

    return body_fn


def _prep_body(degp_ref, x_ref, w1_ref, yp_ref, dinv_ref):
    deg = degp_ref[0, :] + degp_ref[1, :] + 1.0
    dinv = lax.rsqrt(deg)
    xw = jnp.dot(x_ref[...], w1_ref[...], preferred_element_type=jnp.float32)
    yp_ref[...] = xw * dinv[:, None]
    dinv_ref[...] = dinv[:, None]


def _mid_body(up_ref, yp_ref, dinv_ref, b_ref, w_ref, out_ref):
    u = up_ref[0] + up_ref[1] + yp_ref[...]
    h = jnp.maximum(u * dinv_ref[...] + b_ref[...], 0.0)
    out_ref[...] = jnp.dot(h, w_ref[...],
                           preferred_element_type=jnp.float32) * dinv_ref[...]


def _final_body(up_ref, yp_ref, dinv_ref, b_ref, wa_ref, wb_ref,
                go_ref, gd_ref):
    u = up_ref[0] + up_ref[1] + yp_ref[...]
    h = jnp.maximum(u * dinv_ref[...] + b_ref[...], 0.0)
    go_ref[...] = jnp.dot(h, wa_ref[...], preferred_element_type=jnp.float32)
    gd_ref[...] = jnp.dot(h, wb_ref[...], preferred_element_type=jnp.float32)


def _tables_body(dt_ref, tt_ref, wt_ref, ohd_ref, oht_ref, ohw_ref,
                 wta_ref, wtb_ref, wtc_ref, bt_ref, wp1c_ref,
                 modep_ref, wp1d_ref, t_ref, m_ref):
    d1 = jnp.dot(dt_ref[...], wta_ref[...], preferred_element_type=jnp.float32)
    d2 = jnp.dot(tt_ref[...], wtb_ref[...], preferred_element_type=jnp.float32)
    d3 = jnp.dot(wt_ref[...], wtc_ref[...], preferred_element_type=jnp.float32)
    t64 = jnp.dot(ohd_ref[...], d1, preferred_element_type=jnp.float32)
    t64 += jnp.dot(oht_ref[...], d2, preferred_element_type=jnp.float32)
    t64 += jnp.dot(ohw_ref[...], d3, preferred_element_type=jnp.float32)
    t64 = jnp.maximum(t64 + bt_ref[...], 0.0)
    t_ref[...] = jnp.dot(t64, wp1c_ref[...], preferred_element_type=jnp.float32)
    m_ref[...] = jnp.dot(modep_ref[...], wp1d_ref[...],
                         preferred_element_type=jnp.float32)


def _head_body(go_ref, gd_ref, day_ref, time_ref, dow_ref, mode_ref,
               t_ref, m_ref, bp1_ref, wp2_ref, bp2_ref, wp3_ref, bp3_ref,
               wp4_ref, bp4_ref, out_ref):
    cidx = day_ref[...] * 56 + time_ref[...] * 7 + dow_ref[...]
    oht = (cidx == lax.broadcasted_iota(jnp.int32, (RB, 224), 1))
    ohm = (mode_ref[...] == lax.broadcasted_iota(jnp.int32, (RB, 8), 1))
    z = (go_ref[...] + gd_ref[...]
         + jnp.dot(oht.astype(jnp.float32), t_ref[...],
                   preferred_element_type=jnp.float32)
         + jnp.dot(ohm.astype(jnp.float32), m_ref[...],
                   preferred_element_type=jnp.float32)
         + bp1_ref[...])
    z = jnp.maximum(z, 0.0)
    z = jnp.maximum(jnp.dot(z, wp2_ref[...],
                            preferred_element_type=jnp.float32) + bp2_ref[...],
                    0.0)
    z = jnp.maximum(jnp.dot(z, wp3_ref[...],
                            preferred_element_type=jnp.float32) + bp3_ref[...],
                    0.0)
    sc = (jnp.dot(z, wp4_ref[...], preferred_element_type=jnp.float32)
          + bp4_ref[...])
    out_ref[...] = 1.0 / (1.0 + jnp.exp(-sc))


def _prep(degp, xp, W1):
    return pl.pallas_call(
        _prep_body,
        grid=(NP // RB,),
        in_specs=[
            pl.BlockSpec((2, RB), lambda i: (0, i)),
            pl.BlockSpec((RB, DD), lambda i: (i, 0)),
            pl.BlockSpec((DD, HH), lambda i: (0, 0)),
        ],
        out_specs=[
            pl.BlockSpec((RB, HH), lambda i: (i, 0)),
            pl.BlockSpec((RB, 1), lambda i: (i, 0)),
        ],
        out_shape=[
            jax.ShapeDtypeStruct((NP, HH), jnp.float32),
            jax.ShapeDtypeStruct((NP, 1), jnp.float32),
        ],
    )(degp, xp, W1)


def _mid(up, yp, dinv, b, W):
    return pl.pallas_call(
        _mid_body,
        grid=(NP // RB,),
        in_specs=[
            pl.BlockSpec((2, RB, HH), lambda i: (0, i, 0)),
            pl.BlockSpec((RB, HH), lambda i: (i, 0)),
            pl.BlockSpec((RB, 1), lambda i: (i, 0)),
            pl.BlockSpec((1, HH), lambda i: (0, 0)),
            pl.BlockSpec((HH, HH), lambda i: (0, 0)),
        ],
        out_specs=pl.BlockSpec((RB, HH), lambda i: (i, 0)),
        out_shape=jax.ShapeDtypeStruct((NP, HH), jnp.float32),
    )(up, yp, dinv, b, W)


def _final(up, yp, dinv, b, Wa, Wb):
    return pl.pallas_call(
        _final_body,
        grid=(NP // RB,),
        in_specs=[
            pl.BlockSpec((2, RB, HH), lambda i: (0, i, 0)),
            pl.BlockSpec((RB, HH), lambda i: (i, 0)),
            pl.BlockSpec((RB, 1), lambda i: (i, 0)),
            pl.BlockSpec((1, HH), lambda i: (0, 0)),
            pl.BlockSpec((HH, 2 * HH), lambda i: (0, 0)),
            pl.BlockSpec((HH, 2 * HH), lambda i: (0, 0)),
        ],
        out_specs=[
            pl.BlockSpec((RB, 2 * HH), lambda i: (i, 0)),
            pl.BlockSpec((RB, 2 * HH), lambda i: (i, 0)),
        ],
        out_shape=[
            jax.ShapeDtypeStruct((NP, 2 * HH), jnp.float32),
            jax.ShapeDtypeStruct((NP, 2 * HH), jnp.float32),
        ],
    )(up, yp, dinv, b, Wa, Wb)


def _tables(day_tab, time_tab, dow_tab, ohd, oht, ohw, wta, wtb, wtc, bt,
            wp1c, modep, wp1d):
    return pl.pallas_call(
        _tables_body,
        out_shape=[
            jax.ShapeDtypeStruct((224, 2 * HH), jnp.float32),
            jax.ShapeDtypeStruct((8, 2 * HH), jnp.float32),
        ],
    )(day_tab, time_tab, dow_tab, ohd, oht, ohw, wta, wtb, wtc, bt,
      wp1c, modep, wp1d)


def _head(go_g, gd_g, day, time_, dow, mode, t_comb, m_comb,
          bp1, Wp2, bp2, Wp3, bp3, Wp4, bp4):
    return pl.pallas_call(
        _head_body,
        grid=(BB // RB,),
        in_specs=[
            pl.BlockSpec((RB, 2 * HH), lambda i: (i, 0)),
            pl.BlockSpec((RB, 2 * HH), lambda i: (i, 0)),
            pl.BlockSpec((RB, 1), lambda i: (i, 0)),
            pl.BlockSpec((RB, 1), lambda i: (i, 0)),
            pl.BlockSpec((RB, 1), lambda i: (i, 0)),
            pl.BlockSpec((RB, 1), lambda i: (i, 0)),
            pl.BlockSpec((224, 2 * HH), lambda i: (0, 0)),
            pl.BlockSpec((8, 2 * HH), lambda i: (0, 0)),
            pl.BlockSpec((1, 2 * HH), lambda i: (0, 0)),
            pl.BlockSpec((2 * HH, HH), lambda i: (0, 0)),
            pl.BlockSpec((1, HH), lambda i: (0, 0)),
            pl.BlockSpec((HH, HH // 2), lambda i: (0, 0)),
            pl.BlockSpec((1, HH // 2), lambda i: (0, 0)),
            pl.BlockSpec((HH // 2, 1), lambda i: (0, 0)),
            pl.BlockSpec((1, 1), lambda i: (0, 0)),
        ],
        out_specs=pl.BlockSpec((RB, 1), lambda i: (i, 0)),
        out_shape=jax.ShapeDtypeStruct((BB, 1), jnp.float32),
    )(go_g, gd_g, day, time_, dow, mode, t_comb, m_comb,
      bp1, Wp2, bp2, Wp3, bp3, Wp4, bp4)


def kernel(x, edge_index, origin_ids, destination_ids, day_type_ids,
           time_bucket_ids, day_of_week_ids, mode_ids, W1, b1, W2, b2, W3, b3,
           day_tab, time_tab, dow_tab, Wt, bt, mode_tab,
           Wp1, bp1, Wp2, bp2, Wp3, bp3, Wp4, bp4):
    f32 = jnp.float32
    src = edge_index[0]
    dst = edge_index[1]
    npad = EPAD - src.shape[0]
    pad_src = (jnp.arange(npad, dtype=jnp.int32) * 37) % NN
    pad_dst = NN + (jnp.arange(npad, dtype=jnp.int32) % (NP - NN))
    srcp = jnp.concatenate([src, pad_src]).reshape(NW, KE, CH)
    dstp = jnp.concatenate([dst, pad_dst]).reshape(NW, KE, CH)
    xp = jnp.pad(x, ((0, NP - NN), (0, 0)))
    zeros1 = jnp.zeros((NP,), f32)
    zeros2 = jnp.zeros((NP, HH), f32)

    degp = _deg_kernel()(dstp, zeros1)
    yp1, dinv = _prep(degp, xp, W1)
    u1 = _agg_kernel()(yp1, srcp, dstp, zeros2)
    yp2 = _mid(u1, yp1, dinv, b1.reshape(1, HH), W2)
    u2 = _agg_kernel()(yp2, srcp, dstp, zeros2)
    yp3 = _mid(u2, yp2, dinv, b2.reshape(1, HH), W3)
    u3 = _agg_kernel()(yp3, srcp, dstp, zeros2)
    g_o, g_d = _final(u3, yp3, dinv, b3.reshape(1, HH),
                      Wp1[:HH], Wp1[HH:2 * HH])

    car = jnp.arange(224, dtype=jnp.int32)
    ohd = (car[:, None] // 56 == jnp.arange(4)[None, :]).astype(f32)
    oht = (car[:, None] % 56 // 7 == jnp.arange(8)[None, :]).astype(f32)
    ohw = (car[:, None] % 7 == jnp.arange(7)[None, :]).astype(f32)
    modep = jnp.pad(mode_tab, ((0, 8 - mode_tab.shape[0]), (0, 0)))
    t_comb, m_comb = _tables(
        day_tab, time_tab, dow_tab, ohd, oht, ohw,
        Wt[:21], Wt[21:42], Wt[42:63], bt.reshape(1, HH),
        Wp1[2 * HH:3 * HH], modep, Wp1[3 * HH:])

    oid = origin_ids.reshape(NW, KB, CH)
    did = destination_ids.reshape(NW, KB, CH)
    go_g, gd_g = _pair_gather_kernel()(g_o, g_d, oid, did)

    score = _head(
        go_g, gd_g,
        day_type_ids.reshape(BB, 1), time_bucket_ids.reshape(BB, 1),
        day_of_week_ids.reshape(BB, 1), mode_ids.reshape(BB, 1),
        t_comb, m_comb, bp1.reshape(1, 2 * HH), Wp2, bp2.reshape(1, HH),
        Wp3, bp3.reshape(1, HH // 2), Wp4, bp4.reshape(1, 1))
    return score


# speedup vs baseline: 29.5082x; 1.4339x over previous
import functools

import jax
import jax.numpy as jnp
from jax import lax
from jax.experimental import pallas as pl
from jax.experimental.pallas import tpu as pltpu
from jax.experimental.pallas import tpu_sc as plsc

NN = 10000
NP = 10240
DD = 128
HH = 64
BB = 16384
NC, NS, CH = 2, 16, 128
NW = NC * NS
KE = 80
KB = BB // (NW * CH)
EPAD = NW * KE * CH
RB = 512


def _sc_mesh():
    return plsc.VectorSubcoreMesh(core_axis_name="c", subcore_axis_name="s",
                                  num_cores=NC, num_subcores=NS)


@functools.cache
def _deg_kernel():
    @functools.partial(
        pl.kernel,
        out_type=jax.ShapeDtypeStruct((NC, NP), jnp.float32),
        mesh=_sc_mesh(),
        compiler_params=pltpu.CompilerParams(use_tc_tiling_on_sc=False),
        scratch_types=[
            pltpu.VMEM((KE, CH), jnp.int32),
            pltpu.VMEM((CH,), jnp.float32),
            pltpu.VMEM_SHARED((NP,), jnp.float32),
        ],
    )
    def body_fn(dst_hbm, zeros_hbm, out_hbm, dst_v, ones_v, acc_sh):
        c = lax.axis_index("c")
        s = lax.axis_index("s")
        w = c * NS + s
        pltpu.sync_copy(dst_hbm.at[w], dst_v)
        for j in range(CH // 16):
            ones_v[pl.ds(j * 16, 16)] = jnp.ones((16,), jnp.float32)

        @pl.when(s == 0)
        def _():
            pltpu.sync_copy(zeros_hbm, acc_sh)

        plsc.subcore_barrier()

        def body(k, carry):
            pltpu.sync_copy(ones_v, acc_sh.at[dst_v.at[k]], add=True)
            return carry

        lax.fori_loop(0, KE, body, 0)
        plsc.subcore_barrier()

        @pl.when(s == 0)
        def _():
            pltpu.sync_copy(acc_sh, out_hbm.at[c])

    return body_fn


@functools.cache
def _agg_kernel():
    @functools.partial(
        pl.kernel,
        out_type=jax.ShapeDtypeStruct((NC, NP, HH), jnp.float32),
        mesh=_sc_mesh(),
        compiler_params=pltpu.CompilerParams(use_tc_tiling_on_sc=False),
        scratch_types=[
            pltpu.VMEM((KE, CH), jnp.int32),
            pltpu.VMEM((KE, CH), jnp.int32),
            pltpu.VMEM((4, CH, HH), jnp.float32),
            pltpu.VMEM_SHARED((NP, HH), jnp.float32),
            pltpu.SemaphoreType.DMA((4,)),
            pltpu.SemaphoreType.DMA((4,)),
        ],
    )
    def body_fn(yp_hbm, src_hbm, dst_hbm, zeros_hbm, out_hbm,
                src_v, dst_v, rows_v, acc_sh, gsem, ssem):
        c = lax.axis_index("c")
        s = lax.axis_index("s")
        w = c * NS + s
        pltpu.sync_copy(src_hbm.at[w], src_v)
        pltpu.sync_copy(dst_hbm.at[w], dst_v)
        for b in range(3):
            pltpu.async_copy(yp_hbm.at[src_v.at[b]], rows_v.at[b], gsem.at[b])

        @pl.when(s == 0)
        def _():
            pltpu.sync_copy(zeros_hbm, acc_sh)

        plsc.subcore_barrier()

        def body(i, carry):
            for b in range(4):
                k = i * 4 + b
                bn = (b + 3) % 4
                pltpu.make_async_copy(yp_hbm.at[src_v.at[k]], rows_v.at[b],
                                      gsem.at[b]).wait()
                pltpu.async_copy(rows_v.at[b], acc_sh.at[dst_v.at[k]],
                                 ssem.at[b], add=True)

                @pl.when(k >= 1)
                def _():
                    pltpu.make_async_copy(rows_v.at[bn],
                                          acc_sh.at[dst_v.at[k - 1]],
                                          ssem.at[bn]).wait()

                @pl.when(k + 3 < KE)
                def _():
                    pltpu.async_copy(yp_hbm.at[src_v.at[k + 3]], rows_v.at[bn],
                                     gsem.at[bn])
            return carry

        lax.fori_loop(0, KE // 4, body, 0)
        pltpu.make_async_copy(rows_v.at[(KE - 1) % 4],
                              acc_sh.at[dst_v.at[KE - 1]],
                              ssem.at[(KE - 1) % 4]).wait()
        plsc.subcore_barrier()

        @pl.when(s == 0)
        def _():
            pltpu.sync_copy(acc_sh, out_hbm.at[c])

    return body_fn


@functools.cache
def _pair_gather_kernel():
    @functools.partial(
        pl.kernel,
        out_type=[
            jax.ShapeDtypeStruct((BB, 2 * HH), jnp.float32),
            jax.ShapeDtypeStruct((BB, 2 * HH), jnp.float32),
        ],
        mesh=_sc_mesh(),
        compiler_params=pltpu.CompilerParams(use_tc_tiling_on_sc=False),
        scratch_types=[
            pltpu.VMEM((KB, CH), jnp.int32),
            pltpu.VMEM((KB, CH), jnp.int32),
            pltpu.VMEM((CH, 2 * HH), jnp.float32),
            pltpu.SemaphoreType.DMA,
        ],
    )
    def body_fn(go_hbm, gd_hbm, oid_hbm, did_hbm, oo_hbm, od_hbm,
                oi_v, di_v, rows_v, sem):
        c = lax.axis_index("c")
        s = lax.axis_index("s")
        w = c * NS + s
        pltpu.sync_copy(oid_hbm.at[w], oi_v)
        pltpu.sync_copy(did_hbm.at[w], di_v)

        def body(k, carry):
            base = w * (KB * CH) + k * CH
            pltpu.async_copy(go_hbm.at[oi_v.at[k]], rows_v, sem).wait()
            pltpu.sync_copy(rows_v, oo_hbm.at[pl.ds(base, CH)])
            pltpu.async_copy(gd_hbm.at[di_v.at[k]], rows_v, sem).wait()
            pltpu.sync_copy(rows_v, od_hbm.at[pl.ds(base, CH)])
            return carry

        lax.fori_loop(0, KB, body, 0)

    return body_fn


def _prep_body(degp_ref, x_ref, w1_ref, yp_ref, dinv_ref):
    deg = degp_ref[0, :] + degp_ref[1, :] + 1.0
    dinv = lax.rsqrt(deg)
    xw = jnp.dot(x_ref[...], w1_ref[...], preferred_element_type=jnp.float32)
    yp_ref[...] = xw * dinv[:, None]
    dinv_ref[...] = dinv[:, None]


def _mid_body(up_ref, yp_ref, dinv_ref, b_ref, w_ref, out_ref):
    u = up_ref[0] + up_ref[1] + yp_ref[...]
    h = jnp.maximum(u * dinv_ref[...] + b_ref[...], 0.0)
    out_ref[...] = jnp.dot(h, w_ref[...],
                           preferred_element_type=jnp.float32) * dinv_ref[...]


def _final_body(up_ref, yp_ref, dinv_ref, b_ref, wa_ref, wb_ref,
                go_ref, gd_ref):
    u = up_ref[0] + up_ref[1] + yp_ref[...]
    h = jnp.maximum(u * dinv_ref[...] + b_ref[...], 0.0)
    go_ref[...] = jnp.dot(h, wa_ref[...], preferred_element_type=jnp.float32)
    gd_ref[...] = jnp.dot(h, wb_ref[...], preferred_element_type=jnp.float32)


def _tables_body(dt_ref, tt_ref, wt_ref, ohd_ref, oht_ref, ohw_ref,
                 wta_ref, wtb_ref, wtc_ref, bt_ref, wp1c_ref,
                 modep_ref, wp1d_ref, t_ref, m_ref):
    d1 = jnp.dot(dt_ref[...], wta_ref[...], preferred_element_type=jnp.float32)
    d2 = jnp.dot(tt_ref[...], wtb_ref[...], preferred_element_type=jnp.float32)
    d3 = jnp.dot(wt_ref[...], wtc_ref[...], preferred_element_type=jnp.float32)
    t64 = jnp.dot(ohd_ref[...], d1, preferred_element_type=jnp.float32)
    t64 += jnp.dot(oht_ref[...], d2, preferred_element_type=jnp.float32)
    t64 += jnp.dot(ohw_ref[...], d3, preferred_element_type=jnp.float32)
    t64 = jnp.maximum(t64 + bt_ref[...], 0.0)
    t_ref[...] = jnp.dot(t64, wp1c_ref[...], preferred_element_type=jnp.float32)
    m_ref[...] = jnp.dot(modep_ref[...], wp1d_ref[...],
                         preferred_element_type=jnp.float32)


def _head_body(go_ref, gd_ref, day_ref, time_ref, dow_ref, mode_ref,
               t_ref, m_ref, bp1_ref, wp2_ref, bp2_ref, wp3_ref, bp3_ref,
               wp4_ref, bp4_ref, out_ref):
    cidx = day_ref[...] * 56 + time_ref[...] * 7 + dow_ref[...]
    oht = (cidx == lax.broadcasted_iota(jnp.int32, (RB, 224), 1))
    ohm = (mode_ref[...] == lax.broadcasted_iota(jnp.int32, (RB, 8), 1))
    z = (go_ref[...] + gd_ref[...]
         + jnp.dot(oht.astype(jnp.float32), t_ref[...],
                   preferred_element_type=jnp.float32)
         + jnp.dot(ohm.astype(jnp.float32), m_ref[...],
                   preferred_element_type=jnp.float32)
         + bp1_ref[...])
    z = jnp.maximum(z, 0.0)
    z = jnp.maximum(jnp.dot(z, wp2_ref[...],
                            preferred_element_type=jnp.float32) + bp2_ref[...],
                    0.0)
    z = jnp.maximum(jnp.dot(z, wp3_ref[...],
                            preferred_element_type=jnp.float32) + bp3_ref[...],
                    0.0)
    sc = (jnp.dot(z, wp4_ref[...], preferred_element_type=jnp.float32)
          + bp4_ref[...])
    out_ref[...] = 1.0 / (1.0 + jnp.exp(-sc))


def _prep(degp, xp, W1):
    return pl.pallas_call(
        _prep_body,
        grid=(NP // RB,),
        in_specs=[
            pl.BlockSpec((2, RB), lambda i: (0, i)),
            pl.BlockSpec((RB, DD), lambda i: (i, 0)),
            pl.BlockSpec((DD, HH), lambda i: (0, 0)),
        ],
        out_specs=[
            pl.BlockSpec((RB, HH), lambda i: (i, 0)),
            pl.BlockSpec((RB, 1), lambda i: (i, 0)),
        ],
        out_shape=[
            jax.ShapeDtypeStruct((NP, HH), jnp.float32),
            jax.ShapeDtypeStruct((NP, 1), jnp.float32),
        ],
    )(degp, xp, W1)


def _mid(up, yp, dinv, b, W):
    return pl.pallas_call(
        _mid_body,
        grid=(NP // RB,),
        in_specs=[
            pl.BlockSpec((2, RB, HH), lambda i: (0, i, 0)),
            pl.BlockSpec((RB, HH), lambda i: (i, 0)),
            pl.BlockSpec((RB, 1), lambda i: (i, 0)),
            pl.BlockSpec((1, HH), lambda i: (0, 0)),
            pl.BlockSpec((HH, HH), lambda i: (0, 0)),
        ],
        out_specs=pl.BlockSpec((RB, HH), lambda i: (i, 0)),
        out_shape=jax.ShapeDtypeStruct((NP, HH), jnp.float32),
    )(up, yp, dinv, b, W)


def _final(up, yp, dinv, b, Wa, Wb):
    return pl.pallas_call(
        _final_body,
        grid=(NP // RB,),
        in_specs=[
            pl.BlockSpec((2, RB, HH), lambda i: (0, i, 0)),
            pl.BlockSpec((RB, HH), lambda i: (i, 0)),
            pl.BlockSpec((RB, 1), lambda i: (i, 0)),
            pl.BlockSpec((1, HH), lambda i: (0, 0)),
            pl.BlockSpec((HH, 2 * HH), lambda i: (0, 0)),
            pl.BlockSpec((HH, 2 * HH), lambda i: (0, 0)),
        ],
        out_specs=[
            pl.BlockSpec((RB, 2 * HH), lambda i: (i, 0)),
            pl.BlockSpec((RB, 2 * HH), lambda i: (i, 0)),
        ],
        out_shape=[
            jax.ShapeDtypeStruct((NP, 2 * HH), jnp.float32),
            jax.ShapeDtypeStruct((NP, 2 * HH), jnp.float32),
        ],
    )(up, yp, dinv, b, Wa, Wb)


def _tables(day_tab, time_tab, dow_tab, ohd, oht, ohw, wta, wtb, wtc, bt,
            wp1c, modep, wp1d):
    return pl.pallas_call(
        _tables_body,
        out_shape=[
            jax.ShapeDtypeStruct((224, 2 * HH), jnp.float32),
            jax.ShapeDtypeStruct((8, 2 * HH), jnp.float32),
        ],
    )(day_tab, time_tab, dow_tab, ohd, oht, ohw, wta, wtb, wtc, bt,
      wp1c, modep, wp1d)


def _head(go_g, gd_g, day, time_, dow, mode, t_comb, m_comb,
          bp1, Wp2, bp2, Wp3, bp3, Wp4, bp4):
    return pl.pallas_call(
        _head_body,
        grid=(BB // RB,),
        in_specs=[
            pl.BlockSpec((RB, 2 * HH), lambda i: (i, 0)),
            pl.BlockSpec((RB, 2 * HH), lambda i: (i, 0)),
            pl.BlockSpec((RB, 1), lambda i: (i, 0)),
            pl.BlockSpec((RB, 1), lambda i: (i, 0)),
            pl.BlockSpec((RB, 1), lambda i: (i, 0)),
            pl.BlockSpec((RB, 1), lambda i: (i, 0)),
            pl.BlockSpec((224, 2 * HH), lambda i: (0, 0)),
            pl.BlockSpec((8, 2 * HH), lambda i: (0, 0)),
            pl.BlockSpec((1, 2 * HH), lambda i: (0, 0)),
            pl.BlockSpec((2 * HH, HH), lambda i: (0, 0)),
            pl.BlockSpec((1, HH), lambda i: (0, 0)),
            pl.BlockSpec((HH, HH // 2), lambda i: (0, 0)),
            pl.BlockSpec((1, HH // 2), lambda i: (0, 0)),
            pl.BlockSpec((HH // 2, 1), lambda i: (0, 0)),
            pl.BlockSpec((1, 1), lambda i: (0, 0)),
        ],
        out_specs=pl.BlockSpec((RB, 1), lambda i: (i, 0)),
        out_shape=jax.ShapeDtypeStruct((BB, 1), jnp.float32),
    )(go_g, gd_g, day, time_, dow, mode, t_comb, m_comb,
      bp1, Wp2, bp2, Wp3, bp3, Wp4, bp4)


def kernel(x, edge_index, origin_ids, destination_ids, day_type_ids,
           time_bucket_ids, day_of_week_ids, mode_ids, W1, b1, W2, b2, W3, b3,
           day_tab, time_tab, dow_tab, Wt, bt, mode_tab,
           Wp1, bp1, Wp2, bp2, Wp3, bp3, Wp4, bp4):
    f32 = jnp.float32
    src = edge_index[0]
    dst = edge_index[1]
    npad = EPAD - src.shape[0]
    pad_src = (jnp.arange(npad, dtype=jnp.int32) * 37) % NN
    pad_dst = NN + (jnp.arange(npad, dtype=jnp.int32) % (NP - NN))
    srcp = jnp.concatenate([src, pad_src]).reshape(NW, KE, CH)
    dstp = jnp.concatenate([dst, pad_dst]).reshape(NW, KE, CH)
    xp = jnp.pad(x, ((0, NP - NN), (0, 0)))
    zeros1 = jnp.zeros((NP,), f32)
    zeros2 = jnp.zeros((NP, HH), f32)

    degp = _deg_kernel()(dstp, zeros1)
    yp1, dinv = _prep(degp, xp, W1)
    u1 = _agg_kernel()(yp1, srcp, dstp, zeros2)
    yp2 = _mid(u1, yp1, dinv, b1.reshape(1, HH), W2)
    u2 = _agg_kernel()(yp2, srcp, dstp, zeros2)
    yp3 = _mid(u2, yp2, dinv, b2.reshape(1, HH), W3)
    u3 = _agg_kernel()(yp3, srcp, dstp, zeros2)
    g_o, g_d = _final(u3, yp3, dinv, b3.reshape(1, HH),
                      Wp1[:HH], Wp1[HH:2 * HH])

    car = jnp.arange(224, dtype=jnp.int32)
    ohd = (car[:, None] // 56 == jnp.arange(4)[None, :]).astype(f32)
    oht = (car[:, None] % 56 // 7 == jnp.arange(8)[None, :]).astype(f32)
    ohw = (car[:, None] % 7 == jnp.arange(7)[None, :]).astype(f32)
    modep = jnp.pad(mode_tab, ((0, 8 - mode_tab.shape[0]), (0, 0)))
    t_comb, m_comb = _tables(
        day_tab, time_tab, dow_tab, ohd, oht, ohw,
        Wt[:21], Wt[21:42], Wt[42:63], bt.reshape(1, HH),
        Wp1[2 * HH:3 * HH], modep, Wp1[3 * HH:])

    oid = origin_ids.reshape(NW, KB, CH)
    did = destination_ids.reshape(NW, KB, CH)
    go_g, gd_g = _pair_gather_kernel()(g_o, g_d, oid, did)

    score = _head(
        go_g, gd_g,
        day_type_ids.reshape(BB, 1), time_bucket_ids.reshape(BB, 1),
        day_of_week_ids.reshape(BB, 1), mode_ids.reshape(BB, 1),
        t_comb, m_comb, bp1.reshape(1, 2 * HH), Wp2, bp2.reshape(1, HH),
        Wp3, bp3.reshape(1, HH // 2), Wp4, bp4.reshape(1, 1))
    return score


# speedup vs baseline: 30.7185x; 1.0410x over previous
import functools

import jax
import jax.numpy as jnp
from jax import lax
from jax.experimental import pallas as pl
from jax.experimental.pallas import tpu as pltpu
from jax.experimental.pallas import tpu_sc as plsc

NN = 10000
NP = 10240
DD = 128
HH = 64
BB = 16384
NC, NS, CH = 2, 16, 128
NW = NC * NS
KE = 80
KB = BB // (NW * CH)
EPAD = NW * KE * CH
RB = 512
RBH = 2048


def _sc_mesh():
    return plsc.VectorSubcoreMesh(core_axis_name="c", subcore_axis_name="s",
                                  num_cores=NC, num_subcores=NS)


@functools.cache
def _deg_kernel():
    @functools.partial(
        pl.kernel,
        out_type=jax.ShapeDtypeStruct((NC, NP), jnp.float32),
        mesh=_sc_mesh(),
        compiler_params=pltpu.CompilerParams(use_tc_tiling_on_sc=False),
        scratch_types=[
            pltpu.VMEM((KE, CH), jnp.int32),
            pltpu.VMEM((CH,), jnp.float32),
            pltpu.VMEM_SHARED((NP,), jnp.float32),
        ],
    )
    def body_fn(dst_hbm, zeros_hbm, out_hbm, dst_v, ones_v, acc_sh):
        c = lax.axis_index("c")
        s = lax.axis_index("s")
        w = c * NS + s
        pltpu.sync_copy(dst_hbm.at[w], dst_v)
        for j in range(CH // 16):
            ones_v[pl.ds(j * 16, 16)] = jnp.ones((16,), jnp.float32)

        @pl.when(s == 0)
        def _():
            pltpu.sync_copy(zeros_hbm, acc_sh)

        plsc.subcore_barrier()

        def body(k, carry):
            pltpu.sync_copy(ones_v, acc_sh.at[dst_v.at[k]], add=True)
            return carry

        lax.fori_loop(0, KE, body, 0)
        plsc.subcore_barrier()

        @pl.when(s == 0)
        def _():
            pltpu.sync_copy(acc_sh, out_hbm.at[c])

    return body_fn


@functools.cache
def _agg_kernel():
    @functools.partial(
        pl.kernel,
        out_type=jax.ShapeDtypeStruct((NC, NP, HH), jnp.float32),
        mesh=_sc_mesh(),
        compiler_params=pltpu.CompilerParams(use_tc_tiling_on_sc=False),
        scratch_types=[
            pltpu.VMEM((KE, CH), jnp.int32),
            pltpu.VMEM((KE, CH), jnp.int32),
            pltpu.VMEM((4, CH, HH), jnp.float32),
            pltpu.VMEM_SHARED((NP, HH), jnp.float32),
            pltpu.SemaphoreType.DMA((4,)),
            pltpu.SemaphoreType.DMA((4,)),
        ],
    )
    def body_fn(yp_hbm, src_hbm, dst_hbm, zeros_hbm, out_hbm,
                src_v, dst_v, rows_v, acc_sh, gsem, ssem):
        c = lax.axis_index("c")
        s = lax.axis_index("s")
        w = c * NS + s
        pltpu.sync_copy(src_hbm.at[w], src_v)
        pltpu.sync_copy(dst_hbm.at[w], dst_v)
        for b in range(3):
            pltpu.async_copy(yp_hbm.at[src_v.at[b]], rows_v.at[b], gsem.at[b])

        @pl.when(s == 0)
        def _():
            pltpu.sync_copy(zeros_hbm, acc_sh)

        plsc.subcore_barrier()

        def body(i, carry):
            for b in range(4):
                k = i * 4 + b
                bn = (b + 3) % 4
                pltpu.make_async_copy(yp_hbm.at[src_v.at[k]], rows_v.at[b],
                                      gsem.at[b]).wait()
                pltpu.async_copy(rows_v.at[b], acc_sh.at[dst_v.at[k]],
                                 ssem.at[b], add=True)

                @pl.when(k >= 1)
                def _():
                    pltpu.make_async_copy(rows_v.at[bn],
                                          acc_sh.at[dst_v.at[k - 1]],
                                          ssem.at[bn]).wait()

                @pl.when(k + 3 < KE)
                def _():
                    pltpu.async_copy(yp_hbm.at[src_v.at[k + 3]], rows_v.at[bn],
                                     gsem.at[bn])
            return carry

        lax.fori_loop(0, KE // 4, body, 0)
        pltpu.make_async_copy(rows_v.at[(KE - 1) % 4],
                              acc_sh.at[dst_v.at[KE - 1]],
                              ssem.at[(KE - 1) % 4]).wait()
        plsc.subcore_barrier()

        @pl.when(s == 0)
        def _():
            pltpu.sync_copy(acc_sh, out_hbm.at[c])

    return body_fn


@functools.cache
def _pair_gather_kernel():
    @functools.partial(
        pl.kernel,
        out_type=[
            jax.ShapeDtypeStruct((BB, 2 * HH), jnp.float32),
            jax.ShapeDtypeStruct((BB, 2 * HH), jnp.float32),
        ],
        mesh=_sc_mesh(),
        compiler_params=pltpu.CompilerParams(use_tc_tiling_on_sc=False),
        scratch_types=[
            pltpu.VMEM((KB, CH), jnp.int32),
            pltpu.VMEM((KB, CH), jnp.int32),
            pltpu.VMEM((2, CH, 2 * HH), jnp.float32),
            pltpu.VMEM((2, CH, 2 * HH), jnp.float32),
            pltpu.SemaphoreType.DMA((2,)),
            pltpu.SemaphoreType.DMA((2,)),
        ],
    )
    def body_fn(go_hbm, gd_hbm, oid_hbm, did_hbm, oo_hbm, od_hbm,
                oi_v, di_v, ro_v, rd_v, so, sd):
        c = lax.axis_index("c")
        s = lax.axis_index("s")
        w = c * NS + s
        pltpu.sync_copy(oid_hbm.at[w], oi_v)
        pltpu.sync_copy(did_hbm.at[w], di_v)
        pltpu.async_copy(go_hbm.at[oi_v.at[0]], ro_v.at[0], so.at[0])
        pltpu.async_copy(gd_hbm.at[di_v.at[0]], rd_v.at[0], sd.at[0])

        for k in range(KB):
            b, bn = k % 2, (k + 1) % 2
            if k + 1 < KB:
                pltpu.async_copy(go_hbm.at[oi_v.at[k + 1]], ro_v.at[bn],
                                 so.at[bn])
                pltpu.async_copy(gd_hbm.at[di_v.at[k + 1]], rd_v.at[bn],
                                 sd.at[bn])
            base = w * (KB * CH) + k * CH
            pltpu.make_async_copy(go_hbm.at[oi_v.at[k]], ro_v.at[b],
                                  so.at[b]).wait()
            pltpu.sync_copy(ro_v.at[b], oo_hbm.at[pl.ds(base, CH)])
            pltpu.make_async_copy(gd_hbm.at[di_v.at[k]], rd_v.at[b],
                                  sd.at[b]).wait()
            pltpu.sync_copy(rd_v.at[b], od_hbm.at[pl.ds(base, CH)])

    return body_fn


def _prep_body(degp_ref, x_ref, w1_ref, yp_ref, dinv_ref):
    deg = degp_ref[0, :] + degp_ref[1, :] + 1.0
    dinv = lax.rsqrt(deg)
    xw = jnp.dot(x_ref[...], w1_ref[...], preferred_element_type=jnp.float32)
    yp_ref[...] = xw * dinv[:, None]
    dinv_ref[...] = dinv[:, None]


def _mid_body(up_ref, yp_ref, dinv_ref, b_ref, w_ref, out_ref):
    u = up_ref[0] + up_ref[1] + yp_ref[...]
    h = jnp.maximum(u * dinv_ref[...] + b_ref[...], 0.0)
    out_ref[...] = jnp.dot(h, w_ref[...],
                           preferred_element_type=jnp.float32) * dinv_ref[...]


def _final_body(up_ref, yp_ref, dinv_ref, b_ref, wa_ref, wb_ref,
                go_ref, gd_ref):
    u = up_ref[0] + up_ref[1] + yp_ref[...]
    h = jnp.maximum(u * dinv_ref[...] + b_ref[...], 0.0)
    go_ref[...] = jnp.dot(h, wa_ref[...], preferred_element_type=jnp.float32)
    gd_ref[...] = jnp.dot(h, wb_ref[...], preferred_element_type=jnp.float32)


def _tables_body(dt_ref, tt_ref, wt_ref, ohd_ref, oht_ref, ohw_ref,
                 wta_ref, wtb_ref, wtc_ref, bt_ref, wp1c_ref,
                 modep_ref, wp1d_ref, t_ref, m_ref):
    d1 = jnp.dot(dt_ref[...], wta_ref[...], preferred_element_type=jnp.float32)
    d2 = jnp.dot(tt_ref[...], wtb_ref[...], preferred_element_type=jnp.float32)
    d3 = jnp.dot(wt_ref[...], wtc_ref[...], preferred_element_type=jnp.float32)
    t64 = jnp.dot(ohd_ref[...], d1, preferred_element_type=jnp.float32)
    t64 += jnp.dot(oht_ref[...], d2, preferred_element_type=jnp.float32)
    t64 += jnp.dot(ohw_ref[...], d3, preferred_element_type=jnp.float32)
    t64 = jnp.maximum(t64 + bt_ref[...], 0.0)
    t_ref[...] = jnp.dot(t64, wp1c_ref[...], preferred_element_type=jnp.float32)
    m_ref[...] = jnp.dot(modep_ref[...], wp1d_ref[...],
                         preferred_element_type=jnp.float32)


def _head_body(go_ref, gd_ref, day_ref, time_ref, dow_ref, mode_ref,
               t_ref, m_ref, bp1_ref, wp2_ref, bp2_ref, wp3_ref, bp3_ref,
               wp4_ref, bp4_ref, out_ref):
    cidx = day_ref[...] * 56 + time_ref[...] * 7 + dow_ref[...]
    oht = (cidx == lax.broadcasted_iota(jnp.int32, (RBH, 224), 1))
    ohm = (mode_ref[...] == lax.broadcasted_iota(jnp.int32, (RBH, 8), 1))
    z = (go_ref[...] + gd_ref[...]
         + jnp.dot(oht.astype(jnp.float32), t_ref[...],
                   preferred_element_type=jnp.float32)
         + jnp.dot(ohm.astype(jnp.float32), m_ref[...],
                   preferred_element_type=jnp.float32)
         + bp1_ref[...])
    z = jnp.maximum(z, 0.0)
    z = jnp.maximum(jnp.dot(z, wp2_ref[...],
                            preferred_element_type=jnp.float32) + bp2_ref[...],
                    0.0)
    z = jnp.maximum(jnp.dot(z, wp3_ref[...],
                            preferred_element_type=jnp.float32) + bp3_ref[...],
                    0.0)
    sc = (jnp.dot(z, wp4_ref[...], preferred_element_type=jnp.float32)
          + bp4_ref[...])
    out_ref[...] = 1.0 / (1.0 + jnp.exp(-sc))


def _prep(degp, xp, W1):
    return pl.pallas_call(
        _prep_body,
        grid=(NP // RB,),
        in_specs=[
            pl.BlockSpec((2, RB), lambda i: (0, i)),
            pl.BlockSpec((RB, DD), lambda i: (i, 0)),
            pl.BlockSpec((DD, HH), lambda i: (0, 0)),
        ],
        out_specs=[
            pl.BlockSpec((RB, HH), lambda i: (i, 0)),
            pl.BlockSpec((RB, 1), lambda i: (i, 0)),
        ],
        out_shape=[
            jax.ShapeDtypeStruct((NP, HH), jnp.float32),
            jax.ShapeDtypeStruct((NP, 1), jnp.float32),
        ],
    )(degp, xp, W1)


def _mid(up, yp, dinv, b, W):
    return pl.pallas_call(
        _mid_body,
        grid=(NP // RB,),
        in_specs=[
            pl.BlockSpec((2, RB, HH), lambda i: (0, i, 0)),
            pl.BlockSpec((RB, HH), lambda i: (i, 0)),
            pl.BlockSpec((RB, 1), lambda i: (i, 0)),
            pl.BlockSpec((1, HH), lambda i: (0, 0)),
            pl.BlockSpec((HH, HH), lambda i: (0, 0)),
        ],
        out_specs=pl.BlockSpec((RB, HH), lambda i: (i, 0)),
        out_shape=jax.ShapeDtypeStruct((NP, HH), jnp.float32),
    )(up, yp, dinv, b, W)


def _final(up, yp, dinv, b, Wa, Wb):
    return pl.pallas_call(
        _final_body,
        grid=(NP // RB,),
        in_specs=[
            pl.BlockSpec((2, RB, HH), lambda i: (0, i, 0)),
            pl.BlockSpec((RB, HH), lambda i: (i, 0)),
            pl.BlockSpec((RB, 1), lambda i: (i, 0)),
            pl.BlockSpec((1, HH), lambda i: (0, 0)),
            pl.BlockSpec((HH, 2 * HH), lambda i: (0, 0)),
            pl.BlockSpec((HH, 2 * HH), lambda i: (0, 0)),
        ],
        out_specs=[
            pl.BlockSpec((RB, 2 * HH), lambda i: (i, 0)),
            pl.BlockSpec((RB, 2 * HH), lambda i: (i, 0)),
        ],
        out_shape=[
            jax.ShapeDtypeStruct((NP, 2 * HH), jnp.float32),
            jax.ShapeDtypeStruct((NP, 2 * HH), jnp.float32),
        ],
    )(up, yp, dinv, b, Wa, Wb)


def _tables(day_tab, time_tab, dow_tab, ohd, oht, ohw, wta, wtb, wtc, bt,
            wp1c, modep, wp1d):
    return pl.pallas_call(
        _tables_body,
        out_shape=[
            jax.ShapeDtypeStruct((224, 2 * HH), jnp.float32),
            jax.ShapeDtypeStruct((8, 2 * HH), jnp.float32),
        ],
    )(day_tab, time_tab, dow_tab, ohd, oht, ohw, wta, wtb, wtc, bt,
      wp1c, modep, wp1d)


def _head(go_g, gd_g, day, time_, dow, mode, t_comb, m_comb,
          bp1, Wp2, bp2, Wp3, bp3, Wp4, bp4):
    return pl.pallas_call(
        _head_body,
        grid=(BB // RBH,),
        in_specs=[
            pl.BlockSpec((RBH, 2 * HH), lambda i: (i, 0)),
            pl.BlockSpec((RBH, 2 * HH), lambda i: (i, 0)),
            pl.BlockSpec((RBH, 1), lambda i: (i, 0)),
            pl.BlockSpec((RBH, 1), lambda i: (i, 0)),
            pl.BlockSpec((RBH, 1), lambda i: (i, 0)),
            pl.BlockSpec((RBH, 1), lambda i: (i, 0)),
            pl.BlockSpec((224, 2 * HH), lambda i: (0, 0)),
            pl.BlockSpec((8, 2 * HH), lambda i: (0, 0)),
            pl.BlockSpec((1, 2 * HH), lambda i: (0, 0)),
            pl.BlockSpec((2 * HH, HH), lambda i: (0, 0)),
            pl.BlockSpec((1, HH), lambda i: (0, 0)),
            pl.BlockSpec((HH, HH // 2), lambda i: (0, 0)),
            pl.BlockSpec((1, HH // 2), lambda i: (0, 0)),
            pl.BlockSpec((HH // 2, 1), lambda i: (0, 0)),
            pl.BlockSpec((1, 1), lambda i: (0, 0)),
        ],
        out_specs=pl.BlockSpec((RBH, 1), lambda i: (i, 0)),
        out_shape=jax.ShapeDtypeStruct((BB, 1), jnp.float32),
    )(go_g, gd_g, day, time_, dow, mode, t_comb, m_comb,
      bp1, Wp2, bp2, Wp3, bp3, Wp4, bp4)


def kernel(x, edge_index, origin_ids, destination_ids, day_type_ids,
           time_bucket_ids, day_of_week_ids, mode_ids, W1, b1, W2, b2, W3, b3,
           day_tab, time_tab, dow_tab, Wt, bt, mode_tab,
           Wp1, bp1, Wp2, bp2, Wp3, bp3, Wp4, bp4):
    f32 = jnp.float32
    src = edge_index[0]
    dst = edge_index[1]
    npad = EPAD - src.shape[0]
    pad_src = (jnp.arange(npad, dtype=jnp.int32) * 37) % NN
    pad_dst = NN + (jnp.arange(npad, dtype=jnp.int32) % (NP - NN))
    srcp = jnp.concatenate([src, pad_src]).reshape(NW, KE, CH)
    dstp = jnp.concatenate([dst, pad_dst]).reshape(NW, KE, CH)
    xp = jnp.pad(x, ((0, NP - NN), (0, 0)))
    zeros1 = jnp.zeros((NP,), f32)
    zeros2 = jnp.zeros((NP, HH), f32)

    degp = _deg_kernel()(dstp, zeros1)
    yp1, dinv = _prep(degp, xp, W1)
    u1 = _agg_kernel()(yp1, srcp, dstp, zeros2)
    yp2 = _mid(u1, yp1, dinv, b1.reshape(1, HH), W2)
    u2 = _agg_kernel()(yp2, srcp, dstp, zeros2)
    yp3 = _mid(u2, yp2, dinv, b2.reshape(1, HH), W3)
    u3 = _agg_kernel()(yp3, srcp, dstp, zeros2)
    g_o, g_d = _final(u3, yp3, dinv, b3.reshape(1, HH),
                      Wp1[:HH], Wp1[HH:2 * HH])

    car = jnp.arange(224, dtype=jnp.int32)
    ohd = (car[:, None] // 56 == jnp.arange(4)[None, :]).astype(f32)
    oht = (car[:, None] % 56 // 7 == jnp.arange(8)[None, :]).astype(f32)
    ohw = (car[:, None] % 7 == jnp.arange(7)[None, :]).astype(f32)
    modep = jnp.pad(mode_tab, ((0, 8 - mode_tab.shape[0]), (0, 0)))
    t_comb, m_comb = _tables(
        day_tab, time_tab, dow_tab, ohd, oht, ohw,
        Wt[:21], Wt[21:42], Wt[42:63], bt.reshape(1, HH),
        Wp1[2 * HH:3 * HH], modep, Wp1[3 * HH:])

    oid = origin_ids.reshape(NW, KB, CH)
    did = destination_ids.reshape(NW, KB, CH)
    go_g, gd_g = _pair_gather_kernel()(g_o, g_d, oid, did)

    score = _head(
        go_g, gd_g,
        day_type_ids.reshape(BB, 1), time_bucket_ids.reshape(BB, 1),
        day_of_week_ids.reshape(BB, 1), mode_ids.reshape(BB, 1),
        t_comb, m_comb, bp1.reshape(1, 2 * HH), Wp2, bp2.reshape(1, HH),
        Wp3, bp3.reshape(1, HH // 2), Wp4, bp4.reshape(1, 1))
    return score


# speedup vs baseline: 34.3167x; 1.1171x over previous
import functools

import jax
import jax.numpy as jnp
from jax import lax
from jax.experimental import pallas as pl
from jax.experimental.pallas import tpu as pltpu
from jax.experimental.pallas import tpu_sc as plsc

NN = 10000
NP = 10240
DD = 128
HH = 64
BB = 16384
NC, NS, CH = 2, 16, 128
NW = NC * NS
KE = 80
KB = BB // (NW * CH)
EPAD = NW * KE * CH
RB = 512
RBH = 2048


def _sc_mesh():
    return plsc.VectorSubcoreMesh(core_axis_name="c", subcore_axis_name="s",
                                  num_cores=NC, num_subcores=NS)


@functools.cache
def _deg_kernel():
    @functools.partial(
        pl.kernel,
        out_type=jax.ShapeDtypeStruct((NC, NP), jnp.float32),
        mesh=_sc_mesh(),
        compiler_params=pltpu.CompilerParams(use_tc_tiling_on_sc=False),
        scratch_types=[
            pltpu.VMEM((KE, CH), jnp.int32),
            pltpu.VMEM((CH,), jnp.float32),
            pltpu.VMEM_SHARED((NP,), jnp.float32),
        ],
    )
    def body_fn(dst_hbm, zeros_hbm, out_hbm, dst_v, ones_v, acc_sh):
        c = lax.axis_index("c")
        s = lax.axis_index("s")
        w = c * NS + s
        pltpu.sync_copy(dst_hbm.at[w], dst_v)
        for j in range(CH // 16):
            ones_v[pl.ds(j * 16, 16)] = jnp.ones((16,), jnp.float32)

        @pl.when(s == 0)
        def _():
            pltpu.sync_copy(zeros_hbm, acc_sh)

        plsc.subcore_barrier()

        def body(k, carry):
            pltpu.sync_copy(ones_v, acc_sh.at[dst_v.at[k]], add=True)
            return carry

        lax.fori_loop(0, KE, body, 0)
        plsc.subcore_barrier()

        @pl.when(s == 0)
        def _():
            pltpu.sync_copy(acc_sh, out_hbm.at[c])

    return body_fn


@functools.cache
def _agg_kernel():
    @functools.partial(
        pl.kernel,
        out_type=jax.ShapeDtypeStruct((NC, NP, HH), jnp.float32),
        mesh=_sc_mesh(),
        compiler_params=pltpu.CompilerParams(use_tc_tiling_on_sc=False),
        scratch_types=[
            pltpu.VMEM((KE, CH), jnp.int32),
            pltpu.VMEM((KE, CH), jnp.int32),
            pltpu.VMEM((4, CH, HH), jnp.float32),
            pltpu.VMEM_SHARED((NP, HH), jnp.float32),
            pltpu.SemaphoreType.DMA((4,)),
            pltpu.SemaphoreType.DMA((4,)),
        ],
    )
    def body_fn(yp_hbm, src_hbm, dst_hbm, zeros_hbm, out_hbm,
                src_v, dst_v, rows_v, acc_sh, gsem, ssem):
        c = lax.axis_index("c")
        s = lax.axis_index("s")
        w = c * NS + s
        pltpu.sync_copy(src_hbm.at[w], src_v)
        pltpu.sync_copy(dst_hbm.at[w], dst_v)
        for b in range(3):
            pltpu.async_copy(yp_hbm.at[src_v.at[b]], rows_v.at[b], gsem.at[b])

        @pl.when(s == 0)
        def _():
            pltpu.sync_copy(zeros_hbm, acc_sh)

        plsc.subcore_barrier()

        def body(i, carry):
            for b in range(4):
                k = i * 4 + b
                bn = (b + 3) % 4
                pltpu.make_async_copy(yp_hbm.at[src_v.at[k]], rows_v.at[b],
                                      gsem.at[b]).wait()
                pltpu.async_copy(rows_v.at[b], acc_sh.at[dst_v.at[k]],
                                 ssem.at[b], add=True)

                @pl.when(k >= 1)
                def _():
                    pltpu.make_async_copy(rows_v.at[bn],
                                          acc_sh.at[dst_v.at[k - 1]],
                                          ssem.at[bn]).wait()

                @pl.when(k + 3 < KE)
                def _():
                    pltpu.async_copy(yp_hbm.at[src_v.at[k + 3]], rows_v.at[bn],
                                     gsem.at[bn])
            return carry

        lax.fori_loop(0, KE // 4, body, 0)
        pltpu.make_async_copy(rows_v.at[(KE - 1) % 4],
                              acc_sh.at[dst_v.at[KE - 1]],
                              ssem.at[(KE - 1) % 4]).wait()
        plsc.subcore_barrier()

        @pl.when(s == 0)
        def _():
            pltpu.sync_copy(acc_sh, out_hbm.at[c])

    return body_fn


@functools.cache
def _pair_gather_kernel():
    @functools.partial(
        pl.kernel,
        out_type=[
            jax.ShapeDtypeStruct((BB, 2 * HH), jnp.float32),
            jax.ShapeDtypeStruct((BB, 2 * HH), jnp.float32),
        ],
        mesh=_sc_mesh(),
        compiler_params=pltpu.CompilerParams(use_tc_tiling_on_sc=False),
        scratch_types=[
            pltpu.VMEM((KB, CH), jnp.int32),
            pltpu.VMEM((KB, CH), jnp.int32),
            pltpu.VMEM((2, CH, 2 * HH), jnp.float32),
            pltpu.VMEM((2, CH, 2 * HH), jnp.float32),
            pltpu.SemaphoreType.DMA((2,)),
            pltpu.SemaphoreType.DMA((2,)),
        ],
    )
    def body_fn(go_hbm, gd_hbm, oid_hbm, did_hbm, oo_hbm, od_hbm,
                oi_v, di_v, ro_v, rd_v, so, sd):
        c = lax.axis_index("c")
        s = lax.axis_index("s")
        w = c * NS + s
        pltpu.sync_copy(oid_hbm.at[w], oi_v)
        pltpu.sync_copy(did_hbm.at[w], di_v)
        pltpu.async_copy(go_hbm.at[oi_v.at[0]], ro_v.at[0], so.at[0])
        pltpu.async_copy(gd_hbm.at[di_v.at[0]], rd_v.at[0], sd.at[0])

        for k in range(KB):
            b, bn = k % 2, (k + 1) % 2
            if k + 1 < KB:
                pltpu.async_copy(go_hbm.at[oi_v.at[k + 1]], ro_v.at[bn],
                                 so.at[bn])
                pltpu.async_copy(gd_hbm.at[di_v.at[k + 1]], rd_v.at[bn],
                                 sd.at[bn])
            base = w * (KB * CH) + k * CH
            pltpu.make_async_copy(go_hbm.at[oi_v.at[k]], ro_v.at[b],
                                  so.at[b]).wait()
            pltpu.sync_copy(ro_v.at[b], oo_hbm.at[pl.ds(base, CH)])
            pltpu.make_async_copy(gd_hbm.at[di_v.at[k]], rd_v.at[b],
                                  sd.at[b]).wait()
            pltpu.sync_copy(rd_v.at[b], od_hbm.at[pl.ds(base, CH)])

    return body_fn


def _dotf(a, b):
    return jnp.dot(a, b, preferred_element_type=jnp.float32)


def _prep_body(degp_ref, x_ref, w1_ref, se_ref, so_ref, plo_ref, phi_ref,
               yp_ref, dinvp_ref):
    deg = degp_ref[0, :] + degp_ref[1, :] + 1.0
    dinv = lax.rsqrt(deg)
    xw = _dotf(x_ref[...], w1_ref[...])
    y = xw * dinv[:, None]
    yp_ref[...] = (_dotf(_dotf(se_ref[...], y), plo_ref[...])
                   + _dotf(_dotf(so_ref[...], y), phi_ref[...]))
    dc = dinv[:, None] * jnp.ones((1, HH), jnp.float32)
    dinvp_ref[...] = (_dotf(_dotf(se_ref[...], dc), plo_ref[...])
                      + _dotf(_dotf(so_ref[...], dc), phi_ref[...]))


def _mid_body(up_ref, yp_ref, dinvp_ref, bp_ref, wbd_ref, out_ref):
    u = up_ref[0] + up_ref[1] + yp_ref[...]
    h = jnp.maximum(u * dinvp_ref[...] + bp_ref[...], 0.0)
    out_ref[...] = _dotf(h, wbd_ref[...]) * dinvp_ref[...]


def _final_body(up_ref, yp_ref, dinvp_ref, bp_ref, set_ref, sot_ref,
                wae_ref, wao_ref, wbe_ref, wbo_ref, go_ref, gd_ref):
    u = up_ref[0] + up_ref[1] + yp_ref[...]
    h = jnp.maximum(u * dinvp_ref[...] + bp_ref[...], 0.0)
    go_ref[...] = (_dotf(set_ref[...], _dotf(h, wae_ref[...]))
                   + _dotf(sot_ref[...], _dotf(h, wao_ref[...])))
    gd_ref[...] = (_dotf(set_ref[...], _dotf(h, wbe_ref[...]))
                   + _dotf(sot_ref[...], _dotf(h, wbo_ref[...])))


def _tables_body(dt_ref, tt_ref, wt_ref, ohd_ref, oht_ref, ohw_ref,
                 wta_ref, wtb_ref, wtc_ref, bt_ref, wp1c_ref,
                 modep_ref, wp1d_ref, t_ref, m_ref):
    d1 = jnp.dot(dt_ref[...], wta_ref[...], preferred_element_type=jnp.float32)
    d2 = jnp.dot(tt_ref[...], wtb_ref[...], preferred_element_type=jnp.float32)
    d3 = jnp.dot(wt_ref[...], wtc_ref[...], preferred_element_type=jnp.float32)
    t64 = jnp.dot(ohd_ref[...], d1, preferred_element_type=jnp.float32)
    t64 += jnp.dot(oht_ref[...], d2, preferred_element_type=jnp.float32)
    t64 += jnp.dot(ohw_ref[...], d3, preferred_element_type=jnp.float32)
    t64 = jnp.maximum(t64 + bt_ref[...], 0.0)
    t_ref[...] = jnp.dot(t64, wp1c_ref[...], preferred_element_type=jnp.float32)
    m_ref[...] = jnp.dot(modep_ref[...], wp1d_ref[...],
                         preferred_element_type=jnp.float32)


def _head_body(go_ref, gd_ref, day_ref, time_ref, dow_ref, mode_ref,
               t_ref, m_ref, bp1_ref, wp2_ref, bp2_ref, wp3_ref, bp3_ref,
               wp4_ref, bp4_ref, out_ref):
    cidx = day_ref[...] * 56 + time_ref[...] * 7 + dow_ref[...]
    oht = (cidx == lax.broadcasted_iota(jnp.int32, (RBH, 224), 1))
    ohm = (mode_ref[...] == lax.broadcasted_iota(jnp.int32, (RBH, 8), 1))
    z = (go_ref[...] + gd_ref[...]
         + jnp.dot(oht.astype(jnp.float32), t_ref[...],
                   preferred_element_type=jnp.float32)
         + jnp.dot(ohm.astype(jnp.float32), m_ref[...],
                   preferred_element_type=jnp.float32)
         + bp1_ref[...])
    z = jnp.maximum(z, 0.0)
    z = jnp.maximum(jnp.dot(z, wp2_ref[...],
                            preferred_element_type=jnp.float32) + bp2_ref[...],
                    0.0)
    z = jnp.maximum(jnp.dot(z, wp3_ref[...],
                            preferred_element_type=jnp.float32) + bp3_ref[...],
                    0.0)
    sc = (jnp.dot(z, wp4_ref[...], preferred_element_type=jnp.float32)
          + bp4_ref[...])
    out_ref[...] = 1.0 / (1.0 + jnp.exp(-sc))


def _prep(degp, xp, W1, Se, So, Plo, Phi):
    return pl.pallas_call(
        _prep_body,
        grid=(NP // RB,),
        in_specs=[
            pl.BlockSpec((2, RB), lambda i: (0, i)),
            pl.BlockSpec((RB, DD), lambda i: (i, 0)),
            pl.BlockSpec((DD, HH), lambda i: (0, 0)),
            pl.BlockSpec((RB // 2, RB), lambda i: (0, 0)),
            pl.BlockSpec((RB // 2, RB), lambda i: (0, 0)),
            pl.BlockSpec((HH, 2 * HH), lambda i: (0, 0)),
            pl.BlockSpec((HH, 2 * HH), lambda i: (0, 0)),
        ],
        out_specs=[
            pl.BlockSpec((RB // 2, 2 * HH), lambda i: (i, 0)),
            pl.BlockSpec((RB // 2, 2 * HH), lambda i: (i, 0)),
        ],
        out_shape=[
            jax.ShapeDtypeStruct((NP // 2, 2 * HH), jnp.float32),
            jax.ShapeDtypeStruct((NP // 2, 2 * HH), jnp.float32),
        ],
    )(degp, xp, W1, Se, So, Plo, Phi)


def _mid(up, ypp, dinvp, bp, Wbd):
    return pl.pallas_call(
        _mid_body,
        grid=(NP // RB,),
        in_specs=[
            pl.BlockSpec((2, RB // 2, 2 * HH), lambda i: (0, i, 0)),
            pl.BlockSpec((RB // 2, 2 * HH), lambda i: (i, 0)),
            pl.BlockSpec((RB // 2, 2 * HH), lambda i: (i, 0)),
            pl.BlockSpec((1, 2 * HH), lambda i: (0, 0)),
            pl.BlockSpec((2 * HH, 2 * HH), lambda i: (0, 0)),
        ],
        out_specs=pl.BlockSpec((RB // 2, 2 * HH), lambda i: (i, 0)),
        out_shape=jax.ShapeDtypeStruct((NP // 2, 2 * HH), jnp.float32),
    )(up, ypp, dinvp, bp, Wbd)


def _final(up, ypp, dinvp, bp, SeT, SoT, Wae, Wao, Wbe, Wbo):
    return pl.pallas_call(
        _final_body,
        grid=(NP // RB,),
        in_specs=[
            pl.BlockSpec((2, RB // 2, 2 * HH), lambda i: (0, i, 0)),
            pl.BlockSpec((RB // 2, 2 * HH), lambda i: (i, 0)),
            pl.BlockSpec((RB // 2, 2 * HH), lambda i: (i, 0)),
            pl.BlockSpec((1, 2 * HH), lambda i: (0, 0)),
            pl.BlockSpec((RB, RB // 2), lambda i: (0, 0)),
            pl.BlockSpec((RB, RB // 2), lambda i: (0, 0)),
            pl.BlockSpec((2 * HH, 2 * HH), lambda i: (0, 0)),
            pl.BlockSpec((2 * HH, 2 * HH), lambda i: (0, 0)),
            pl.BlockSpec((2 * HH, 2 * HH), lambda i: (0, 0)),
            pl.BlockSpec((2 * HH, 2 * HH), lambda i: (0, 0)),
        ],
        out_specs=[
            pl.BlockSpec((RB, 2 * HH), lambda i: (i, 0)),
            pl.BlockSpec((RB, 2 * HH), lambda i: (i, 0)),
        ],
        out_shape=[
            jax.ShapeDtypeStruct((NP, 2 * HH), jnp.float32),
            jax.ShapeDtypeStruct((NP, 2 * HH), jnp.float32),
        ],
    )(up, ypp, dinvp, bp, SeT, SoT, Wae, Wao, Wbe, Wbo)


def _tables(day_tab, time_tab, dow_tab, ohd, oht, ohw, wta, wtb, wtc, bt,
            wp1c, modep, wp1d):
    return pl.pallas_call(
        _tables_body,
        out_shape=[
            jax.ShapeDtypeStruct((224, 2 * HH), jnp.float32),
            jax.ShapeDtypeStruct((8, 2 * HH), jnp.float32),
        ],
    )(day_tab, time_tab, dow_tab, ohd, oht, ohw, wta, wtb, wtc, bt,
      wp1c, modep, wp1d)


def _head(go_g, gd_g, day, time_, dow, mode, t_comb, m_comb,
          bp1, Wp2, bp2, Wp3, bp3, Wp4, bp4):
    return pl.pallas_call(
        _head_body,
        grid=(BB // RBH,),
        in_specs=[
            pl.BlockSpec((RBH, 2 * HH), lambda i: (i, 0)),
            pl.BlockSpec((RBH, 2 * HH), lambda i: (i, 0)),
            pl.BlockSpec((RBH, 1), lambda i: (i, 0)),
            pl.BlockSpec((RBH, 1), lambda i: (i, 0)),
            pl.BlockSpec((RBH, 1), lambda i: (i, 0)),
            pl.BlockSpec((RBH, 1), lambda i: (i, 0)),
            pl.BlockSpec((224, 2 * HH), lambda i: (0, 0)),
            pl.BlockSpec((8, 2 * HH), lambda i: (0, 0)),
            pl.BlockSpec((1, 2 * HH), lambda i: (0, 0)),
            pl.BlockSpec((2 * HH, HH), lambda i: (0, 0)),
            pl.BlockSpec((1, HH), lambda i: (0, 0)),
            pl.BlockSpec((HH, HH // 2), lambda i: (0, 0)),
            pl.BlockSpec((1, HH // 2), lambda i: (0, 0)),
            pl.BlockSpec((HH // 2, 1), lambda i: (0, 0)),
            pl.BlockSpec((1, 1), lambda i: (0, 0)),
        ],
        out_specs=pl.BlockSpec((RBH, 1), lambda i: (i, 0)),
        out_shape=jax.ShapeDtypeStruct((BB, 1), jnp.float32),
    )(go_g, gd_g, day, time_, dow, mode, t_comb, m_comb,
      bp1, Wp2, bp2, Wp3, bp3, Wp4, bp4)


def kernel(x, edge_index, origin_ids, destination_ids, day_type_ids,
           time_bucket_ids, day_of_week_ids, mode_ids, W1, b1, W2, b2, W3, b3,
           day_tab, time_tab, dow_tab, Wt, bt, mode_tab,
           Wp1, bp1, Wp2, bp2, Wp3, bp3, Wp4, bp4):
    f32 = jnp.float32
    src = edge_index[0]
    dst = edge_index[1]
    npad = EPAD - src.shape[0]
    pad_src = (jnp.arange(npad, dtype=jnp.int32) * 37) % NN
    pad_dst = NN + (jnp.arange(npad, dtype=jnp.int32) % (NP - NN))
    srcp = jnp.concatenate([src, pad_src]).reshape(NW, KE, CH)
    dstp = jnp.concatenate([dst, pad_dst]).reshape(NW, KE, CH)
    xp = jnp.pad(x, ((0, NP - NN), (0, 0)))
    zeros1 = jnp.zeros((NP,), f32)
    zeros2 = jnp.zeros((NP, HH), f32)

    eye2 = jnp.eye(2, dtype=f32)
    W2bd = jnp.kron(eye2, W2)
    W3bd = jnp.kron(eye2, W3)
    b1p = jnp.tile(b1.reshape(1, HH), (1, 2))
    b2p = jnp.tile(b2.reshape(1, HH), (1, 2))
    b3p = jnp.tile(b3.reshape(1, HH), (1, 2))
    pr = jnp.arange(RB // 2)
    cr = jnp.arange(RB)
    Se = (cr[None, :] == 2 * pr[:, None]).astype(f32)
    So = (cr[None, :] == 2 * pr[:, None] + 1).astype(f32)
    eyeh = jnp.eye(HH, dtype=f32)
    zh = jnp.zeros((HH, HH), f32)
    Plo = jnp.concatenate([eyeh, zh], axis=1)
    Phi = jnp.concatenate([zh, eyeh], axis=1)
    Wa, Wb = Wp1[:HH], Wp1[HH:2 * HH]
    z2h = jnp.zeros((HH, 2 * HH), f32)
    Wae = jnp.concatenate([Wa, z2h], axis=0)
    Wao = jnp.concatenate([z2h, Wa], axis=0)
    Wbe = jnp.concatenate([Wb, z2h], axis=0)
    Wbo = jnp.concatenate([z2h, Wb], axis=0)

    degp = _deg_kernel()(dstp, zeros1)
    ypp1, dinvp = _prep(degp, xp, W1, Se, So, Plo, Phi)
    u1 = _agg_kernel()(ypp1.reshape(NP, HH), srcp, dstp, zeros2)
    ypp2 = _mid(u1.reshape(NC, NP // 2, 2 * HH), ypp1, dinvp, b1p, W2bd)
    u2 = _agg_kernel()(ypp2.reshape(NP, HH), srcp, dstp, zeros2)
    ypp3 = _mid(u2.reshape(NC, NP // 2, 2 * HH), ypp2, dinvp, b2p, W3bd)
    u3 = _agg_kernel()(ypp3.reshape(NP, HH), srcp, dstp, zeros2)
    g_o, g_d = _final(u3.reshape(NC, NP // 2, 2 * HH), ypp3, dinvp, b3p,
                      Se.T, So.T, Wae, Wao, Wbe, Wbo)

    car = jnp.arange(224, dtype=jnp.int32)
    ohd = (car[:, None] // 56 == jnp.arange(4)[None, :]).astype(f32)
    oht = (car[:, None] % 56 // 7 == jnp.arange(8)[None, :]).astype(f32)
    ohw = (car[:, None] % 7 == jnp.arange(7)[None, :]).astype(f32)
    modep = jnp.pad(mode_tab, ((0, 8 - mode_tab.shape[0]), (0, 0)))
    t_comb, m_comb = _tables(
        day_tab, time_tab, dow_tab, ohd, oht, ohw,
        Wt[:21], Wt[21:42], Wt[42:63], bt.reshape(1, HH),
        Wp1[2 * HH:3 * HH], modep, Wp1[3 * HH:])

    oid = origin_ids.reshape(NW, KB, CH)
    did = destination_ids.reshape(NW, KB, CH)
    go_g, gd_g = _pair_gather_kernel()(g_o, g_d, oid, did)

    score = _head(
        go_g, gd_g,
        day_type_ids.reshape(BB, 1), time_bucket_ids.reshape(BB, 1),
        day_of_week_ids.reshape(BB, 1), mode_ids.reshape(BB, 1),
        t_comb, m_comb, bp1.reshape(1, 2 * HH), Wp2, bp2.reshape(1, HH),
        Wp3, bp3.reshape(1, HH // 2), Wp4, bp4.reshape(1, 1))
    return score


# speedup vs baseline: 34.3273x; 1.0003x over previous
import functools

import jax
import jax.numpy as jnp
from jax import lax
from jax.experimental import pallas as pl
from jax.experimental.pallas import tpu as pltpu
from jax.experimental.pallas import tpu_sc as plsc

NN = 10000
NP = 10240
DD = 128
HH = 64
BB = 16384
NC, NS, CH = 2, 16, 128
NW = NC * NS
KE = 80
KB = BB // (NW * CH)
EPAD = NW * KE * CH
RB = 512
RBH = 2048


def _sc_mesh():
    return plsc.VectorSubcoreMesh(core_axis_name="c", subcore_axis_name="s",
                                  num_cores=NC, num_subcores=NS)


@functools.cache
def _deg_kernel():
    @functools.partial(
        pl.kernel,
        out_type=jax.ShapeDtypeStruct((NC, NP), jnp.float32),
        mesh=_sc_mesh(),
        compiler_params=pltpu.CompilerParams(use_tc_tiling_on_sc=False),
        scratch_types=[
            pltpu.VMEM((KE, CH), jnp.int32),
            pltpu.VMEM((CH,), jnp.float32),
            pltpu.VMEM_SHARED((NP,), jnp.float32),
        ],
    )
    def body_fn(dst_hbm, zeros_hbm, out_hbm, dst_v, ones_v, acc_sh):
        c = lax.axis_index("c")
        s = lax.axis_index("s")
        w = c * NS + s
        pltpu.sync_copy(dst_hbm.at[w], dst_v)
        for j in range(CH // 16):
            ones_v[pl.ds(j * 16, 16)] = jnp.ones((16,), jnp.float32)

        @pl.when(s == 0)
        def _():
            pltpu.sync_copy(zeros_hbm, acc_sh)

        plsc.subcore_barrier()

        def body(k, carry):
            pltpu.sync_copy(ones_v, acc_sh.at[dst_v.at[k]], add=True)
            return carry

        lax.fori_loop(0, KE, body, 0)
        plsc.subcore_barrier()

        @pl.when(s == 0)
        def _():
            pltpu.sync_copy(acc_sh, out_hbm.at[c])

    return body_fn


@functools.cache
def _agg_kernel():
    @functools.partial(
        pl.kernel,
        out_type=jax.ShapeDtypeStruct((NC, NP, HH), jnp.float32),
        mesh=_sc_mesh(),
        compiler_params=pltpu.CompilerParams(use_tc_tiling_on_sc=False),
        scratch_types=[
            pltpu.VMEM((KE, CH), jnp.int32),
            pltpu.VMEM((KE, CH), jnp.int32),
            pltpu.VMEM((4, CH, HH), jnp.float32),
            pltpu.VMEM_SHARED((NP, HH), jnp.float32),
            pltpu.SemaphoreType.DMA((4,)),
            pltpu.SemaphoreType.DMA((4,)),
        ],
    )
    def body_fn(yp_hbm, src_hbm, dst_hbm, zeros_hbm, out_hbm,
                src_v, dst_v, rows_v, acc_sh, gsem, ssem):
        c = lax.axis_index("c")
        s = lax.axis_index("s")
        w = c * NS + s
        pltpu.sync_copy(src_hbm.at[w], src_v)
        pltpu.sync_copy(dst_hbm.at[w], dst_v)
        for b in range(3):
            pltpu.async_copy(yp_hbm.at[src_v.at[b]], rows_v.at[b], gsem.at[b])

        @pl.when(s == 0)
        def _():
            pltpu.sync_copy(zeros_hbm, acc_sh)

        plsc.subcore_barrier()

        def body(i, carry):
            for b in range(4):
                k = i * 4 + b
                bn = (b + 3) % 4
                pltpu.make_async_copy(yp_hbm.at[src_v.at[k]], rows_v.at[b],
                                      gsem.at[b]).wait()
                pltpu.async_copy(rows_v.at[b], acc_sh.at[dst_v.at[k]],
                                 ssem.at[b], add=True)

                @pl.when(k >= 1)
                def _():
                    pltpu.make_async_copy(rows_v.at[bn],
                                          acc_sh.at[dst_v.at[k - 1]],
                                          ssem.at[bn]).wait()

                @pl.when(k + 3 < KE)
                def _():
                    pltpu.async_copy(yp_hbm.at[src_v.at[k + 3]], rows_v.at[bn],
                                     gsem.at[bn])
            return carry

        lax.fori_loop(0, KE // 4, body, 0)
        pltpu.make_async_copy(rows_v.at[(KE - 1) % 4],
                              acc_sh.at[dst_v.at[KE - 1]],
                              ssem.at[(KE - 1) % 4]).wait()
        plsc.subcore_barrier()

        @pl.when(s == 0)
        def _():
            pltpu.sync_copy(acc_sh, out_hbm.at[c])

    return body_fn


@functools.cache
def _pair_gather_kernel():
    @functools.partial(
        pl.kernel,
        out_type=[
            jax.ShapeDtypeStruct((BB, 2 * HH), jnp.float32),
            jax.ShapeDtypeStruct((BB, 2 * HH), jnp.float32),
        ],
        mesh=_sc_mesh(),
        compiler_params=pltpu.CompilerParams(use_tc_tiling_on_sc=False),
        scratch_types=[
            pltpu.VMEM((KB, CH), jnp.int32),
            pltpu.VMEM((KB, CH), jnp.int32),
            pltpu.VMEM((2, CH, 2 * HH), jnp.float32),
            pltpu.VMEM((2, CH, 2 * HH), jnp.float32),
            pltpu.SemaphoreType.DMA((2,)),
            pltpu.SemaphoreType.DMA((2,)),
        ],
    )
    def body_fn(go_hbm, gd_hbm, oid_hbm, did_hbm, oo_hbm, od_hbm,
                oi_v, di_v, ro_v, rd_v, so, sd):
        c = lax.axis_index("c")
        s = lax.axis_index("s")
        w = c * NS + s
        pltpu.sync_copy(oid_hbm.at[w], oi_v)
        pltpu.sync_copy(did_hbm.at[w], di_v)
        pltpu.async_copy(go_hbm.at[oi_v.at[0]], ro_v.at[0], so.at[0])
        pltpu.async_copy(gd_hbm.at[di_v.at[0]], rd_v.at[0], sd.at[0])

        for k in range(KB):
            b, bn = k % 2, (k + 1) % 2
            if k + 1 < KB:
                pltpu.async_copy(go_hbm.at[oi_v.at[k + 1]], ro_v.at[bn],
                                 so.at[bn])
                pltpu.async_copy(gd_hbm.at[di_v.at[k + 1]], rd_v.at[bn],
                                 sd.at[bn])
            base = w * (KB * CH) + k * CH
            pltpu.make_async_copy(go_hbm.at[oi_v.at[k]], ro_v.at[b],
                                  so.at[b]).wait()
            pltpu.sync_copy(ro_v.at[b], oo_hbm.at[pl.ds(base, CH)])
            pltpu.make_async_copy(gd_hbm.at[di_v.at[k]], rd_v.at[b],
                                  sd.at[b]).wait()
            pltpu.sync_copy(rd_v.at[b], od_hbm.at[pl.ds(base, CH)])

    return body_fn


def _dotf(a, b):
    return jnp.dot(a, b, preferred_element_type=jnp.float32)


def _prep_body(degp_ref, x_ref, w1_ref, se_ref, so_ref, plo_ref, phi_ref,
               yp_ref, dinvp_ref):
    deg = degp_ref[0, :] + degp_ref[1, :] + 1.0
    dinv = lax.rsqrt(deg)
    xw = _dotf(x_ref[...], w1_ref[...])
    y = xw * dinv[:, None]
    ye = _dotf(se_ref[...], y)
    yo = _dotf(so_ref[...], y)
    yp_ref[...] = _dotf(ye, plo_ref[...]) + _dotf(yo, phi_ref[...])
    dc = dinv[:, None] * jnp.ones((1, HH), jnp.float32)
    dinvp_ref[...] = (_dotf(_dotf(se_ref[...], dc), plo_ref[...])
                      + _dotf(_dotf(so_ref[...], dc), phi_ref[...]))


def _mid_body(up_ref, yp_ref, dinvp_ref, bp_ref, wbd_ref, out_ref):
    u = up_ref[0] + up_ref[1] + yp_ref[...]
    h = jnp.maximum(u * dinvp_ref[...] + bp_ref[...], 0.0)
    out_ref[...] = _dotf(h, wbd_ref[...]) * dinvp_ref[...]


def _final_body(up_ref, yp_ref, dinvp_ref, bp_ref, set_ref, sot_ref,
                wae_ref, wao_ref, wbe_ref, wbo_ref, go_ref, gd_ref):
    u = up_ref[0] + up_ref[1] + yp_ref[...]
    h = jnp.maximum(u * dinvp_ref[...] + bp_ref[...], 0.0)
    go = (_dotf(set_ref[...], _dotf(h, wae_ref[...]))
          + _dotf(sot_ref[...], _dotf(h, wao_ref[...])))
    gd = (_dotf(set_ref[...], _dotf(h, wbe_ref[...]))
          + _dotf(sot_ref[...], _dotf(h, wbo_ref[...])))
    go_ref[...] = go.reshape(RB // 8, 8, 2 * HH)
    gd_ref[...] = gd.reshape(RB // 8, 8, 2 * HH)


def _tables_body(dt_ref, tt_ref, wt_ref, ohd_ref, oht_ref, ohw_ref,
                 wta_ref, wtb_ref, wtc_ref, bt_ref, wp1c_ref,
                 modep_ref, wp1d_ref, t_ref, m_ref):
    d1 = jnp.dot(dt_ref[...], wta_ref[...], preferred_element_type=jnp.float32)
    d2 = jnp.dot(tt_ref[...], wtb_ref[...], preferred_element_type=jnp.float32)
    d3 = jnp.dot(wt_ref[...], wtc_ref[...], preferred_element_type=jnp.float32)
    t64 = jnp.dot(ohd_ref[...], d1, preferred_element_type=jnp.float32)
    t64 += jnp.dot(oht_ref[...], d2, preferred_element_type=jnp.float32)
    t64 += jnp.dot(ohw_ref[...], d3, preferred_element_type=jnp.float32)
    t64 = jnp.maximum(t64 + bt_ref[...], 0.0)
    t_ref[...] = jnp.dot(t64, wp1c_ref[...], preferred_element_type=jnp.float32)
    m_ref[...] = jnp.dot(modep_ref[...], wp1d_ref[...],
                         preferred_element_type=jnp.float32)


def _head_body(go_ref, gd_ref, day_ref, time_ref, dow_ref, mode_ref,
               t_ref, m_ref, bp1_ref, wp2_ref, bp2_ref, wp3_ref, bp3_ref,
               wp4_ref, bp4_ref, out_ref):
    cidx = day_ref[...] * 56 + time_ref[...] * 7 + dow_ref[...]
    oht = (cidx == lax.broadcasted_iota(jnp.int32, (RBH, 224), 1))
    ohm = (mode_ref[...] == lax.broadcasted_iota(jnp.int32, (RBH, 8), 1))
    z = (go_ref[...].reshape(RBH, 2 * HH) + gd_ref[...].reshape(RBH, 2 * HH)
         + jnp.dot(oht.astype(jnp.float32), t_ref[...],
                   preferred_element_type=jnp.float32)
         + jnp.dot(ohm.astype(jnp.float32), m_ref[...],
                   preferred_element_type=jnp.float32)
         + bp1_ref[...])
    z = jnp.maximum(z, 0.0)
    z = jnp.maximum(jnp.dot(z, wp2_ref[...],
                            preferred_element_type=jnp.float32) + bp2_ref[...],
                    0.0)
    z = jnp.maximum(jnp.dot(z, wp3_ref[...],
                            preferred_element_type=jnp.float32) + bp3_ref[...],
                    0.0)
    sc = (jnp.dot(z, wp4_ref[...], preferred_element_type=jnp.float32)
          + bp4_ref[...])
    out_ref[...] = 1.0 / (1.0 + jnp.exp(-sc))


def _prep(degp, xp, W1, Se, So, Plo, Phi):
    return pl.pallas_call(
        _prep_body,
        grid=(NP // RB,),
        in_specs=[
            pl.BlockSpec((2, RB), lambda i: (0, i)),
            pl.BlockSpec((RB, DD), lambda i: (i, 0)),
            pl.BlockSpec((DD, HH), lambda i: (0, 0)),
            pl.BlockSpec((RB // 2, RB), lambda i: (0, 0)),
            pl.BlockSpec((RB // 2, RB), lambda i: (0, 0)),
            pl.BlockSpec((HH, 2 * HH), lambda i: (0, 0)),
            pl.BlockSpec((HH, 2 * HH), lambda i: (0, 0)),
        ],
        out_specs=[
            pl.BlockSpec((RB // 2, 2 * HH), lambda i: (i, 0)),
            pl.BlockSpec((RB // 2, 2 * HH), lambda i: (i, 0)),
        ],
        out_shape=[
            jax.ShapeDtypeStruct((NP // 2, 2 * HH), jnp.float32),
            jax.ShapeDtypeStruct((NP // 2, 2 * HH), jnp.float32),
        ],
    )(degp, xp, W1, Se, So, Plo, Phi)


def _mid(up, ypp, dinvp, bp, Wbd):
    return pl.pallas_call(
        _mid_body,
        grid=(NP // RB,),
        in_specs=[
            pl.BlockSpec((2, RB // 2, 2 * HH), lambda i: (0, i, 0)),
            pl.BlockSpec((RB // 2, 2 * HH), lambda i: (i, 0)),
            pl.BlockSpec((RB // 2, 2 * HH), lambda i: (i, 0)),
            pl.BlockSpec((1, 2 * HH), lambda i: (0, 0)),
            pl.BlockSpec((2 * HH, 2 * HH), lambda i: (0, 0)),
        ],
        out_specs=pl.BlockSpec((RB // 2, 2 * HH), lambda i: (i, 0)),
        out_shape=jax.ShapeDtypeStruct((NP // 2, 2 * HH), jnp.float32),
    )(up, ypp, dinvp, bp, Wbd)


def _final(up, ypp, dinvp, bp, SeT, SoT, Wae, Wao, Wbe, Wbo):
    return pl.pallas_call(
        _final_body,
        grid=(NP // RB,),
        in_specs=[
            pl.BlockSpec((2, RB // 2, 2 * HH), lambda i: (0, i, 0)),
            pl.BlockSpec((RB // 2, 2 * HH), lambda i: (i, 0)),
            pl.BlockSpec((RB // 2, 2 * HH), lambda i: (i, 0)),
            pl.BlockSpec((1, 2 * HH), lambda i: (0, 0)),
            pl.BlockSpec((RB, RB // 2), lambda i: (0, 0)),
            pl.BlockSpec((RB, RB // 2), lambda i: (0, 0)),
            pl.BlockSpec((2 * HH, 2 * HH), lambda i: (0, 0)),
            pl.BlockSpec((2 * HH, 2 * HH), lambda i: (0, 0)),
            pl.BlockSpec((2 * HH, 2 * HH), lambda i: (0, 0)),
            pl.BlockSpec((2 * HH, 2 * HH), lambda i: (0, 0)),
        ],
        out_specs=[
            pl.BlockSpec((RB // 8, 8, 2 * HH), lambda i: (i, 0, 0)),
            pl.BlockSpec((RB // 8, 8, 2 * HH), lambda i: (i, 0, 0)),
        ],
        out_shape=[
            jax.ShapeDtypeStruct((NP // 8, 8, 2 * HH), jnp.float32),
            jax.ShapeDtypeStruct((NP // 8, 8, 2 * HH), jnp.float32),
        ],
    )(up, ypp, dinvp, bp, SeT, SoT, Wae, Wao, Wbe, Wbo)


def _tables(day_tab, time_tab, dow_tab, ohd, oht, ohw, wta, wtb, wtc, bt,
            wp1c, modep, wp1d):
    return pl.pallas_call(
        _tables_body,
        out_shape=[
            jax.ShapeDtypeStruct((224, 2 * HH), jnp.float32),
            jax.ShapeDtypeStruct((8, 2 * HH), jnp.float32),
        ],
    )(day_tab, time_tab, dow_tab, ohd, oht, ohw, wta, wtb, wtc, bt,
      wp1c, modep, wp1d)


def _head(go_g, gd_g, day, time_, dow, mode, t_comb, m_comb,
          bp1, Wp2, bp2, Wp3, bp3, Wp4, bp4):
    return pl.pallas_call(
        _head_body,
        grid=(BB // RBH,),
        in_specs=[
            pl.BlockSpec((RBH // 8, 8, 2 * HH), lambda i: (i, 0, 0)),
            pl.BlockSpec((RBH // 8, 8, 2 * HH), lambda i: (i, 0, 0)),
            pl.BlockSpec((RBH, 1), lambda i: (i, 0)),
            pl.BlockSpec((RBH, 1), lambda i: (i, 0)),
            pl.BlockSpec((RBH, 1), lambda i: (i, 0)),
            pl.BlockSpec((RBH, 1), lambda i: (i, 0)),
            pl.BlockSpec((224, 2 * HH), lambda i: (0, 0)),
            pl.BlockSpec((8, 2 * HH), lambda i: (0, 0)),
            pl.BlockSpec((1, 2 * HH), lambda i: (0, 0)),
            pl.BlockSpec((2 * HH, HH), lambda i: (0, 0)),
            pl.BlockSpec((1, HH), lambda i: (0, 0)),
            pl.BlockSpec((HH, HH // 2), lambda i: (0, 0)),
            pl.BlockSpec((1, HH // 2), lambda i: (0, 0)),
            pl.BlockSpec((HH // 2, 1), lambda i: (0, 0)),
            pl.BlockSpec((1, 1), lambda i: (0, 0)),
        ],
        out_specs=pl.BlockSpec((RBH, 1), lambda i: (i, 0)),
        out_shape=jax.ShapeDtypeStruct((BB, 1), jnp.float32),
    )(go_g, gd_g, day, time_, dow, mode, t_comb, m_comb,
      bp1, Wp2, bp2, Wp3, bp3, Wp4, bp4)


def kernel(x, edge_index, origin_ids, destination_ids, day_type_ids,
           time_bucket_ids, day_of_week_ids, mode_ids, W1, b1, W2, b2, W3, b3,
           day_tab, time_tab, dow_tab, Wt, bt, mode_tab,
           Wp1, bp1, Wp2, bp2, Wp3, bp3, Wp4, bp4):
    f32 = jnp.float32
    src = edge_index[0]
    dst = edge_index[1]
    npad = EPAD - src.shape[0]
    pad_src = (jnp.arange(npad, dtype=jnp.int32) * 37) % NN
    pad_dst = NN + (jnp.arange(npad, dtype=jnp.int32) % (NP - NN))
    srcp = jnp.concatenate([src, pad_src]).reshape(NW, KE, CH)
    dstp = jnp.concatenate([dst, pad_dst]).reshape(NW, KE, CH)
    xp = jnp.pad(x, ((0, NP - NN), (0, 0)))
    zeros1 = jnp.zeros((NP,), f32)
    zeros2 = jnp.zeros((NP, HH), f32)

    eye2 = jnp.eye(2, dtype=f32)
    W2bd = jnp.kron(eye2, W2)
    W3bd = jnp.kron(eye2, W3)
    b1p = jnp.tile(b1.reshape(1, HH), (1, 2))
    b2p = jnp.tile(b2.reshape(1, HH), (1, 2))
    b3p = jnp.tile(b3.reshape(1, HH), (1, 2))
    pr = jnp.arange(RB // 2)
    cr = jnp.arange(RB)
    Se = (cr[None, :] == 2 * pr[:, None]).astype(f32)
    So = (cr[None, :] == 2 * pr[:, None] + 1).astype(f32)
    eyeh = jnp.eye(HH, dtype=f32)
    zh = jnp.zeros((HH, HH), f32)
    Plo = jnp.concatenate([eyeh, zh], axis=1)
    Phi = jnp.concatenate([zh, eyeh], axis=1)
    Wa, Wb = Wp1[:HH], Wp1[HH:2 * HH]
    z2h = jnp.zeros((HH, 2 * HH), f32)
    Wae = jnp.concatenate([Wa, z2h], axis=0)
    Wao = jnp.concatenate([z2h, Wa], axis=0)
    Wbe = jnp.concatenate([Wb, z2h], axis=0)
    Wbo = jnp.concatenate([z2h, Wb], axis=0)

    degp = _deg_kernel()(dstp, zeros1)
    ypp1, dinvp = _prep(degp, xp, W1, Se, So, Plo, Phi)
    u1 = _agg_kernel()(ypp1.reshape(NP, HH), srcp, dstp, zeros2)
    ypp2 = _mid(u1.reshape(NC, NP // 2, 2 * HH), ypp1, dinvp, b1p, W2bd)
    u2 = _agg_kernel()(ypp2.reshape(NP, HH), srcp, dstp, zeros2)
    ypp3 = _mid(u2.reshape(NC, NP // 2, 2 * HH), ypp2, dinvp, b2p, W3bd)
    u3 = _agg_kernel()(ypp3.reshape(NP, HH), srcp, dstp, zeros2)
    g_o, g_d = _final(u3.reshape(NC, NP // 2, 2 * HH), ypp3, dinvp, b3p,
                      Se.T, So.T, Wae, Wao, Wbe, Wbo)

    car = jnp.arange(224, dtype=jnp.int32)
    ohd = (car[:, None] // 56 == jnp.arange(4)[None, :]).astype(f32)
    oht = (car[:, None] % 56 // 7 == jnp.arange(8)[None, :]).astype(f32)
    ohw = (car[:, None] % 7 == jnp.arange(7)[None, :]).astype(f32)
    modep = jnp.pad(mode_tab, ((0, 8 - mode_tab.shape[0]), (0, 0)))
    t_comb, m_comb = _tables(
        day_tab, time_tab, dow_tab, ohd, oht, ohw,
        Wt[:21], Wt[21:42], Wt[42:63], bt.reshape(1, HH),
        Wp1[2 * HH:3 * HH], modep, Wp1[3 * HH:])

    oid = origin_ids.reshape(NW, KB, CH)
    did = destination_ids.reshape(NW, KB, CH)
    go_g, gd_g = _pair_gather_kernel()(g_o.reshape(NP, 2 * HH),
                                       g_d.reshape(NP, 2 * HH), oid, did)

    score = _head(
        go_g.reshape(BB // 8, 8, 2 * HH), gd_g.reshape(BB // 8, 8, 2 * HH),
        day_type_ids.reshape(BB, 1), time_bucket_ids.reshape(BB, 1),
        day_of_week_ids.reshape(BB, 1), mode_ids.reshape(BB, 1),
        t_comb, m_comb, bp1.reshape(1, 2 * HH), Wp2, bp2.reshape(1, HH),
        Wp3, bp3.reshape(1, HH // 2), Wp4, bp4.reshape(1, 1))
    return score
